# baseline (device time: 64512 ns/iter reference)
import jax
import jax.numpy as jnp
from jax import lax
from jax.experimental import pallas as pl
from jax.experimental.pallas import tpu as pltpu

N_DEV = 16


def kernel(x, Win0, Wout0, Win1, Wout1, Win2, Wout2):
    b, d_sh = x.shape
    h = Win0.shape[1]
    rows = b // N_DEV

    f32 = jnp.float32
    bf16 = jnp.bfloat16

    def body(x_ref, win0_ref, wout0_ref, win1_ref, wout1_ref, win2_ref,
             wout2_ref, out_ref, partial_ref, recva_ref, hfull_ref,
             sa_sems, ra_sems, sb_sems, rb_sems):
        my = lax.axis_index("i")
        wins = [win0_ref, win1_ref, win2_ref]
        wouts = [wout0_ref, wout1_ref, wout2_ref]

        def send_a(o):
            tgt = lax.rem(my + o, N_DEV)
            r = pltpu.make_async_remote_copy(
                src_ref=partial_ref.at[pl.ds(tgt * rows, rows)],
                dst_ref=recva_ref.at[N_DEV - o],
                send_sem=sa_sems.at[o],
                recv_sem=ra_sems.at[N_DEV - o],
                device_id=(tgt,),
                device_id_type=pl.DeviceIdType.MESH,
            )
            r.start()
            return r

        def send_b(o):
            tgt = lax.rem(my + o, N_DEV)
            r = pltpu.make_async_remote_copy(
                src_ref=hfull_ref.at[pl.ds(my * rows, rows)],
                dst_ref=hfull_ref.at[pl.ds(my * rows, rows)],
                send_sem=sb_sems.at[o],
                recv_sem=rb_sems.at[N_DEV - o],
                device_id=(tgt,),
                device_id_type=pl.DeviceIdType.MESH,
            )
            r.start()
            return r

        def wait_recv_a(o):
            pltpu.make_async_remote_copy(
                src_ref=recva_ref.at[o],
                dst_ref=recva_ref.at[o],
                send_sem=sa_sems.at[o],
                recv_sem=ra_sems.at[o],
                device_id=(my,),
                device_id_type=pl.DeviceIdType.MESH,
            ).wait_recv()

        def wait_recv_b(o, src_dev):
            pltpu.make_async_remote_copy(
                src_ref=hfull_ref.at[pl.ds(src_dev * rows, rows)],
                dst_ref=hfull_ref.at[pl.ds(src_dev * rows, rows)],
                send_sem=sb_sems.at[o],
                recv_sem=rb_sems.at[o],
                device_id=(my,),
                device_id_type=pl.DeviceIdType.MESH,
            ).wait_recv()

        partial_ref[:, :] = jnp.dot(
            x_ref[:, :].astype(bf16), wins[0][:, :].astype(bf16),
            preferred_element_type=f32,
        ).astype(bf16)
        sends_a = {o: send_a(o) for o in range(1, N_DEV)}
        sends_b = None
        pown = partial_ref[pl.ds(my * rows, rows), :].astype(f32)

        for layer in range(3):
            last = layer == 2

            acc = pown
            for o in range(1, N_DEV):
                wait_recv_a(o)
                acc = acc + recva_ref[o].astype(f32)
            hmine = jnp.maximum(acc, 0.0)
            hfull_ref[pl.ds(my * rows, rows), :] = hmine.astype(bf16)

            if sends_b is not None:
                for o in range(1, N_DEV):
                    sends_b[o].wait_send()
            sends_b = {o: send_b(o) for o in range(1, N_DEV)}

            wout_b = wouts[layer][:, :].astype(bf16)
            xnb_my = jnp.dot(hmine.astype(bf16), wout_b,
                             preferred_element_type=f32)
            if not last:
                win_next_b = wins[layer + 1][:, :].astype(bf16)
                pown = jnp.dot(xnb_my.astype(bf16), win_next_b,
                               preferred_element_type=f32)
            else:
                out_ref[pl.ds(my * rows, rows), :] = xnb_my

            new_sends_a = {}
            for o in range(1, N_DEV):
                s = lax.rem(my + o, N_DEV)
                wait_recv_b(o, s)
                hslice = hfull_ref[pl.ds(s * rows, rows), :]
                xnb = jnp.dot(hslice, wout_b, preferred_element_type=f32)
                if not last:
                    pb = jnp.dot(xnb.astype(bf16), win_next_b,
                                 preferred_element_type=f32).astype(bf16)
                    sends_a[o].wait_send()
                    partial_ref[pl.ds(s * rows, rows), :] = pb
                    new_sends_a[o] = send_a(o)
                else:
                    sends_a[o].wait_send()
                    out_ref[pl.ds(s * rows, rows), :] = xnb
            sends_a = new_sends_a

        for o in range(1, N_DEV):
            sends_b[o].wait_send()

    return pl.pallas_call(
        body,
        out_shape=jax.ShapeDtypeStruct((b, d_sh), jnp.float32),
        in_specs=[pl.BlockSpec(memory_space=pltpu.VMEM)] * 7,
        out_specs=pl.BlockSpec(memory_space=pltpu.VMEM),
        scratch_shapes=[
            pltpu.VMEM((b, h), bf16),
            pltpu.VMEM((N_DEV, rows, h), bf16),
            pltpu.VMEM((b, h), bf16),
            pltpu.SemaphoreType.DMA((N_DEV,)),
            pltpu.SemaphoreType.DMA((N_DEV,)),
            pltpu.SemaphoreType.DMA((N_DEV,)),
            pltpu.SemaphoreType.DMA((N_DEV,)),
        ],
    )(x, Win0, Wout0, Win1, Wout1, Win2, Wout2)
